# baseline (speedup 1.0000x reference)
import jax
import jax.numpy as jnp
from jax import lax
from jax.experimental import pallas as pl
from jax.experimental.pallas import tpu as pltpu
from jax.experimental.pallas import tpu_sc as plsc

NC = 2
NS = 16
NW = NC * NS
LANES = 16

BATCH = 16384
LABELS = 50
EMBED = 64
LPAD = 128

BW = BATCH // NW
ROWS = BW * LABELS
CHUNK = 128
NCHUNK = ROWS // CHUNK
NBUF = 4
QROWS = 128
NQ = BW // QROWS


def _sc_body(idx_hbm, w_hbm, out_hbm, raw_v, idx_v, acc_v,
             b0, b1, b2, b3, s0, s1, s2, s3):
  bufs = (b0, b1, b2, b3)
  sems = (s0, s1, s2, s3)

  wid = lax.axis_index("s") * NC + lax.axis_index("c")

  lane = lax.iota(jnp.int32, LANES)
  tail_mask = lane < (LABELS - 48)

  for q in range(NQ):
    pltpu.sync_copy(
        idx_hbm.at[pl.ds(wid * (BW * LPAD) + q * (QROWS * LPAD),
                         QROWS * LPAD)],
        raw_v)

    @pl.loop(0, QROWS, unroll=4)
    def _compact(r):
      src = r * LPAD
      dst = (q * QROWS + r) * LABELS
      for off in (0, 16, 32):
        v = raw_v[pl.ds(src + off, LANES)]
        plsc.store_scatter(idx_v, [lane + (dst + off)], v)
      v = raw_v[pl.ds(src + 48, LANES)]
      plsc.store_scatter(idx_v, [lane + (dst + 48)], v, mask=tail_mask)

  for b in range(NBUF):
    pltpu.async_copy(w_hbm.at[idx_v.at[pl.ds(b * CHUNK, CHUNK)]],
                     bufs[b], sems[b])

  zero = jnp.zeros((LANES,), jnp.float32)

  @pl.loop(0, BW, unroll=4)
  def _zero(r):
    for c in range(EMBED // LANES):
      acc_v[r, pl.ds(c * LANES, LANES)] = zero

  @pl.loop(0, NCHUNK, step=NBUF)
  def _main(t):
    for b in range(NBUF):
      tt = t + b
      buf = bufs[b]
      sem = sems[b]
      pltpu.make_async_copy(
          w_hbm.at[idx_v.at[pl.ds(tt * CHUNK, CHUNK)]], buf, sem).wait()

      base = tt * CHUNK

      @pl.loop(0, CHUNK, unroll=8)
      def _accum(i):
        brow = (base + i) // LABELS
        for c in range(EMBED // LANES):
          v = buf[i, pl.ds(c * LANES, LANES)]
          plsc.addupdate(acc_v.at[brow, pl.ds(c * LANES, LANES)], v)

      nxt = tt + NBUF

      @pl.when(nxt < NCHUNK)
      def _():
        pltpu.async_copy(
            w_hbm.at[idx_v.at[pl.ds(nxt * CHUNK, CHUNK)]], buf, sem)

  pltpu.sync_copy(acc_v, out_hbm.at[pl.ds(wid * BW, BW)])


@jax.jit
def _run(idx_flat, weight):
  mesh = plsc.VectorSubcoreMesh(
      core_axis_name="c", subcore_axis_name="s",
      num_cores=NC, num_subcores=NS)
  f = pl.kernel(
      _sc_body,
      out_type=jax.ShapeDtypeStruct((BATCH, EMBED), jnp.float32),
      mesh=mesh,
      scratch_types=[
          pltpu.VMEM((QROWS * LPAD,), jnp.int32),
          pltpu.VMEM((ROWS,), jnp.int32),
          pltpu.VMEM((BW, EMBED), jnp.float32),
      ] + [pltpu.VMEM((CHUNK, EMBED), jnp.float32)] * NBUF
        + [pltpu.SemaphoreType.DMA] * NBUF,
      compiler_params=pltpu.CompilerParams(use_tc_tiling_on_sc=False,
                                           needs_layout_passes=False),
  )
  return f(idx_flat, weight)


def kernel(inputs, weight):
  idx = inputs.astype(jnp.int32)
  idx_pad = jnp.pad(idx, ((0, 0), (0, LPAD - LABELS)))
  return _run(idx_pad.reshape(BATCH * LPAD), weight)


# speedup vs baseline: 1.2897x; 1.2897x over previous
import jax
import jax.numpy as jnp
from jax import lax
from jax.experimental import pallas as pl
from jax.experimental.pallas import tpu as pltpu
from jax.experimental.pallas import tpu_sc as plsc

NC = 2
NS = 16
NW = NC * NS
LANES = 16

BATCH = 16384
LABELS = 50
EMBED = 64
LPAD = 128

BW = BATCH // NW
ROWS = BW * LABELS
CHUNK = 128
NCHUNK = ROWS // CHUNK
NSUB = BW // CHUNK
NBUF = 4
QROWS = 128
PB = 512


def _pad_body(x_ref, o_ref):
  o_ref[:, pl.ds(0, LABELS)] = x_ref[...]


def _sc_body(idx_hbm, w_hbm, out_hbm, raw_v, idx_v, acc_v,
             b0, b1, b2, b3, s0, s1, s2, s3):
  bufs = (b0, b1, b2, b3)
  sems = (s0, s1, s2, s3)

  wid = lax.axis_index("s") * NC + lax.axis_index("c")

  lane = lax.iota(jnp.int32, LANES)
  lane_bw = lane * BW
  tail_mask = lane < (LABELS - 48)

  for q in range(BW // QROWS):
    pltpu.sync_copy(
        idx_hbm.at[pl.ds(wid * (BW * LPAD) + q * (QROWS * LPAD),
                         QROWS * LPAD)],
        raw_v)

    @pl.loop(0, QROWS, unroll=4)
    def _compact(r):
      src = r * LPAD
      b = q * QROWS + r
      for off in (0, 16, 32):
        v = raw_v[pl.ds(src + off, LANES)]
        plsc.store_scatter(idx_v, [lane_bw + (off * BW + b)], v)
      v = raw_v[pl.ds(src + 48, LANES)]
      plsc.store_scatter(idx_v, [lane_bw + (48 * BW + b)], v,
                         mask=tail_mask)

  for b in range(NBUF):
    pltpu.async_copy(w_hbm.at[idx_v.at[pl.ds(b * CHUNK, CHUNK)]],
                     bufs[b], sems[b])

  zero = jnp.zeros((LANES,), jnp.float32)

  @pl.loop(0, BW, unroll=4)
  def _zero(r):
    for c in range(EMBED // LANES):
      acc_v[r, pl.ds(c * LANES, LANES)] = zero

  @pl.loop(0, NCHUNK, step=NBUF)
  def _main(t):
    for b in range(NBUF):
      tt = t + b
      buf = bufs[b]
      sem = sems[b]
      pltpu.make_async_copy(
          w_hbm.at[idx_v.at[pl.ds(tt * CHUNK, CHUNK)]], buf, sem).wait()

      base = (tt & (NSUB - 1)) * CHUNK

      @pl.loop(0, CHUNK, unroll=8)
      def _accum(i):
        row = base + i
        for c in range(EMBED // LANES):
          v = buf[i, pl.ds(c * LANES, LANES)]
          plsc.addupdate(acc_v.at[row, pl.ds(c * LANES, LANES)], v)

      nxt = tt + NBUF

      @pl.when(nxt < NCHUNK)
      def _():
        pltpu.async_copy(
            w_hbm.at[idx_v.at[pl.ds(nxt * CHUNK, CHUNK)]], buf, sem)

  pltpu.sync_copy(acc_v, out_hbm.at[pl.ds(wid * BW, BW)])


@jax.jit
def _run(idx, weight):
  idx_pad = pl.pallas_call(
      _pad_body,
      grid=(BATCH // PB,),
      in_specs=[pl.BlockSpec((PB, LABELS), lambda i: (i, 0))],
      out_specs=pl.BlockSpec((PB, LPAD), lambda i: (i, 0)),
      out_shape=jax.ShapeDtypeStruct((BATCH, LPAD), jnp.int32),
  )(idx)
  idx_flat = idx_pad.reshape(BATCH * LPAD)

  mesh = plsc.VectorSubcoreMesh(
      core_axis_name="c", subcore_axis_name="s",
      num_cores=NC, num_subcores=NS)
  f = pl.kernel(
      _sc_body,
      out_type=jax.ShapeDtypeStruct((BATCH, EMBED), jnp.float32),
      mesh=mesh,
      scratch_types=[
          pltpu.VMEM((QROWS * LPAD,), jnp.int32),
          pltpu.VMEM((ROWS,), jnp.int32),
          pltpu.VMEM((BW, EMBED), jnp.float32),
      ] + [pltpu.VMEM((CHUNK, EMBED), jnp.float32)] * NBUF
        + [pltpu.SemaphoreType.DMA] * NBUF,
      compiler_params=pltpu.CompilerParams(use_tc_tiling_on_sc=False,
                                           needs_layout_passes=False),
  )
  return f(idx_flat, weight)


def kernel(inputs, weight):
  return _run(inputs.astype(jnp.int32), weight)


# speedup vs baseline: 1.3228x; 1.0257x over previous
import jax
import jax.numpy as jnp
from jax import lax
from jax.experimental import pallas as pl
from jax.experimental.pallas import tpu as pltpu
from jax.experimental.pallas import tpu_sc as plsc

NC = 2
NS = 16
NW = NC * NS
LANES = 16

BATCH = 16384
LABELS = 50
EMBED = 64
EPAD = 128

BW = BATCH // NW
CHUNK = 128
NSUB = BW // CHUNK
NCHUNK = NSUB * LABELS
NBUF = 4
PB = 1024


def _sc_body(idx_hbm, w_hbm, out_hbm, idx_v, acc_v,
             b0, b1, b2, b3, s0, s1, s2, s3):
  bufs = (b0, b1, b2, b3)
  sems = (s0, s1, s2, s3)

  wid = lax.axis_index("s") * NC + lax.axis_index("c")

  pltpu.sync_copy(idx_hbm.at[wid], idx_v)

  for b in range(NBUF):
    pltpu.async_copy(w_hbm.at[idx_v.at[b]], bufs[b], sems[b])

  zero = jnp.zeros((LANES,), jnp.float32)

  @pl.loop(0, BW, unroll=4)
  def _zero(r):
    for c in range(EMBED // LANES):
      acc_v[r, pl.ds(c * LANES, LANES)] = zero

  @pl.loop(0, NCHUNK, step=NBUF)
  def _main(t):
    for b in range(NBUF):
      tt = t + b
      buf = bufs[b]
      sem = sems[b]
      pltpu.make_async_copy(w_hbm.at[idx_v.at[tt]], buf, sem).wait()

      base = (tt & (NSUB - 1)) * CHUNK

      @pl.loop(0, CHUNK, unroll=8)
      def _accum(i):
        row = base + i
        for c in range(EMBED // LANES):
          v = buf[i, pl.ds(c * LANES, LANES)]
          plsc.addupdate(acc_v.at[row, pl.ds(c * LANES, LANES)], v)

      nxt = tt + NBUF

      @pl.when(nxt < NCHUNK)
      def _():
        pltpu.async_copy(w_hbm.at[idx_v.at[nxt]], buf, sem)

  pltpu.sync_copy(acc_v, out_hbm.at[pl.ds(wid * BW, BW), pl.ds(0, EMBED)])


def _slice_body(x_ref, o_ref):
  o_ref[...] = x_ref[:, pl.ds(0, EMBED)]


@jax.jit
def _run(idx_r, weight):
  mesh = plsc.VectorSubcoreMesh(
      core_axis_name="c", subcore_axis_name="s",
      num_cores=NC, num_subcores=NS)
  f = pl.kernel(
      _sc_body,
      out_type=jax.ShapeDtypeStruct((BATCH, EPAD), jnp.float32),
      mesh=mesh,
      scratch_types=[
          pltpu.VMEM((NCHUNK, CHUNK), jnp.int32),
          pltpu.VMEM((BW, EMBED), jnp.float32),
      ] + [pltpu.VMEM((CHUNK, EMBED), jnp.float32)] * NBUF
        + [pltpu.SemaphoreType.DMA] * NBUF,
      compiler_params=pltpu.CompilerParams(use_tc_tiling_on_sc=False,
                                           needs_layout_passes=False),
  )
  wide = f(idx_r, weight)
  return pl.pallas_call(
      _slice_body,
      grid=(BATCH // PB,),
      in_specs=[pl.BlockSpec((PB, EPAD), lambda i: (i, 0))],
      out_specs=pl.BlockSpec((PB, EMBED), lambda i: (i, 0)),
      out_shape=jax.ShapeDtypeStruct((BATCH, EMBED), jnp.float32),
  )(wide)


def kernel(inputs, weight):
  idx = inputs.astype(jnp.int32)
  idx_r = idx.reshape(NW, NSUB, CHUNK, LABELS).transpose(0, 3, 1, 2)
  idx_r = idx_r.reshape(NW, NCHUNK, CHUNK)
  return _run(idx_r, weight)
